# baseline (speedup 1.0000x reference)
import functools

import jax
import jax.numpy as jnp
from jax import lax
from jax.experimental import pallas as pl
from jax.experimental.pallas import tpu as pltpu
from jax.experimental.pallas import tpu_sc as plsc

VOCAB = 100000
DOC = 100000
DIM = 64
BATCH = 1024
CTX = 20

NC = 2
NS = 16
NW = NC * NS
B_PER_W = BATCH // NW
IDX_PER_W = B_PER_W * CTX
GCHUNK = 128
LANES = 16


def _sc_hidden_body(word_hbm, doc_hbm, ctx_hbm, did_hbm, out_hbm,
                    ctx_idx, doc_idx, ctx_rows, doc_rows, hid, sem):
    wid = lax.axis_index("s") * NC + lax.axis_index("c")
    base = wid * B_PER_W
    ibase = wid * IDX_PER_W

    pltpu.sync_copy(ctx_hbm.at[pl.ds(ibase, IDX_PER_W)], ctx_idx)
    pltpu.sync_copy(did_hbm.at[pl.ds(base, B_PER_W)], doc_idx)

    copies = []
    for k in range(IDX_PER_W // GCHUNK):
        copies.append(pltpu.async_copy(
            word_hbm.at[ctx_idx.at[pl.ds(k * GCHUNK, GCHUNK)]],
            ctx_rows.at[pl.ds(k * GCHUNK, GCHUNK), :], sem))
    copies.append(pltpu.async_copy(doc_hbm.at[doc_idx], doc_rows, sem))
    for c in copies:
        c.wait()

    inv = jnp.float32(1.0 / CTX)

    def row(i, carry):
        for c in range(DIM // LANES):
            sl = pl.ds(c * LANES, LANES)
            acc = ctx_rows[i * CTX, sl]
            for j in range(1, CTX):
                acc = acc + ctx_rows[i * CTX + j, sl]
            hid[i, sl] = acc * inv + doc_rows[i, sl]
        return carry

    lax.fori_loop(0, B_PER_W, row, 0, unroll=False)

    pltpu.sync_copy(hid, out_hbm.at[pl.ds(base, B_PER_W)])


@functools.partial(jax.jit, static_argnames=())
def _sc_hidden(word_emb, doc_emb, ctx_flat, doc_id):
    mesh = plsc.VectorSubcoreMesh(core_axis_name="c", subcore_axis_name="s")
    return pl.kernel(
        _sc_hidden_body,
        mesh=mesh,
        compiler_params=pltpu.CompilerParams(use_tc_tiling_on_sc=False),
        out_type=jax.ShapeDtypeStruct((BATCH, DIM), jnp.float32),
        scratch_types=[
            pltpu.VMEM((IDX_PER_W,), jnp.int32),
            pltpu.VMEM((B_PER_W,), jnp.int32),
            pltpu.VMEM((IDX_PER_W, DIM), jnp.float32),
            pltpu.VMEM((B_PER_W, DIM), jnp.float32),
            pltpu.VMEM((B_PER_W, DIM), jnp.float32),
            pltpu.SemaphoreType.DMA,
        ],
    )(word_emb, doc_emb, ctx_flat, doc_id)


BLKV = 2048


def _proj_body(h_ref, w_ref, b_ref, out_ref):
    out_ref[...] = lax.dot_general(
        h_ref[...], w_ref[...],
        dimension_numbers=(((1,), (1,)), ((), ())),
        preferred_element_type=jnp.float32,
    ) + b_ref[...]


def _tc_logits(hidden, W, b2d):
    grid = (pl.cdiv(VOCAB, BLKV),)
    return pl.pallas_call(
        _proj_body,
        grid=grid,
        in_specs=[
            pl.BlockSpec((BATCH, DIM), lambda j: (0, 0)),
            pl.BlockSpec((BLKV, DIM), lambda j: (j, 0)),
            pl.BlockSpec((1, BLKV), lambda j: (0, j)),
        ],
        out_specs=pl.BlockSpec((BATCH, BLKV), lambda j: (0, j)),
        out_shape=jax.ShapeDtypeStruct((BATCH, VOCAB), jnp.float32),
    )(hidden, W, b2d)


def kernel(doc_id, context_ids, word_emb, doc_emb, W, b):
    ctx_flat = context_ids.reshape(-1).astype(jnp.int32)
    did = doc_id.astype(jnp.int32)
    hidden = _sc_hidden(word_emb, doc_emb, ctx_flat, did)
    return _tc_logits(hidden, W, b.reshape(1, VOCAB))
